# baseline (speedup 1.0000x reference)
import jax
import jax.numpy as jnp
import numpy as np
from jax import lax
from jax.experimental import pallas as pl
from jax.experimental.pallas import tpu as pltpu

MIN_LEVEL = 3
MAX_LEVEL = 7
NUM_SCALES = 3
ASPECTS = [(1.0, 1.0), (1.4, 0.7), (0.7, 1.4)]
ANCHOR_SCALE = 4.0
IMAGE_SIZE = 512
NUM_CLASSES = 90
ALPHA = 0.25
GAMMA = 1.5
DELTA = 0.1
BOX_LOSS_WEIGHT = 50.0
MATCH_THRESHOLD = 0.5


def _gen_anchor_boxes():
    boxes_all = []
    for level in range(MIN_LEVEL, MAX_LEVEL + 1):
        stride = 2 ** level
        boxes_level = []
        for octave in range(NUM_SCALES):
            for (ax, ay) in ASPECTS:
                base = ANCHOR_SCALE * stride * 2.0 ** (octave / float(NUM_SCALES))
                ah2 = base * ay / 2.0
                aw2 = base * ax / 2.0
                x = np.arange(stride / 2.0, IMAGE_SIZE, stride)
                y = np.arange(stride / 2.0, IMAGE_SIZE, stride)
                xv, yv = np.meshgrid(x, y)
                b = np.stack([yv - ah2, xv - aw2, yv + ah2, xv + aw2], axis=-1).reshape(-1, 4)
                boxes_level.append(b)
        boxes_all.append(np.stack(boxes_level, axis=1).reshape(-1, 4))
    return np.concatenate(boxes_all, axis=0).astype(np.float32)


_ANCHOR_BOXES_T = np.ascontiguousarray(_gen_anchor_boxes().T)

BLK = 1584


def _loss_block(cls_ref, box_ref, ancT_ref, gtb_ref, gtl_ref,
                total_ref, clsl_ref, boxl_ref, acc_ref):
    b = pl.program_id(0)
    i = pl.program_id(1)
    nb = pl.num_programs(0)
    ni = pl.num_programs(1)

    @pl.when(jnp.logical_and(b == 0, i == 0))
    def _init():
        acc_ref[0] = 0.0
        acc_ref[1] = 0.0
        acc_ref[2] = 0.0

    ancT = ancT_ref[:, 0, 0, :]
    gtb = gtb_ref[0]
    g = gtb.shape[0]
    g_y0 = gtb[:, 0:1]
    g_x0 = gtb[:, 1:2]
    g_y1 = gtb[:, 2:3]
    g_x1 = gtb[:, 3:4]
    a_y0 = ancT[0:1, :]
    a_x0 = ancT[1:2, :]
    a_y1 = ancT[2:3, :]
    a_x1 = ancT[3:4, :]

    ymin = jnp.maximum(g_y0, a_y0)
    xmin = jnp.maximum(g_x0, a_x0)
    ymax = jnp.minimum(g_y1, a_y1)
    xmax = jnp.minimum(g_x1, a_x1)
    inter = jnp.maximum(ymax - ymin, 0.0) * jnp.maximum(xmax - xmin, 0.0)
    area_g = (g_y1 - g_y0) * (g_x1 - g_x0)
    area_a = (a_y1 - a_y0) * (a_x1 - a_x0)
    union = area_g + area_a - inter
    iou = inter / jnp.maximum(union, 1e-8)

    max_iou = jnp.max(iou, axis=0, keepdims=True)
    idx = lax.broadcasted_iota(jnp.int32, iou.shape, 0)
    arg = jnp.min(jnp.where(iou >= max_iou, idx, g), axis=0, keepdims=True)
    m = (idx == arg).astype(jnp.float32)

    posf = (max_iou >= MATCH_THRESHOLD).astype(jnp.float32)

    matched = lax.dot_general(jnp.transpose(gtb, (1, 0)), m,
                              (((1,), (0,)), ((), ())),
                              preferred_element_type=jnp.float32)
    lab = gtl_ref[0]
    cls_iota = lax.broadcasted_iota(jnp.int32, (g, NUM_CLASSES), 1)
    lab1h = (cls_iota == lab).astype(jnp.float32)
    onehot = lax.dot_general(m * posf, lab1h, (((0,), (0,)), ((), ())),
                             preferred_element_type=jnp.float32)

    eps = 1e-8
    m_y0 = matched[0:1, :]
    m_x0 = matched[1:2, :]
    m_y1 = matched[2:3, :]
    m_x1 = matched[3:4, :]
    ya = (a_y0 + a_y1) * 0.5
    xa = (a_x0 + a_x1) * 0.5
    ha = jnp.maximum(a_y1 - a_y0, eps)
    wa = jnp.maximum(a_x1 - a_x0, eps)
    yc = (m_y0 + m_y1) * 0.5
    xc = (m_x0 + m_x1) * 0.5
    h = jnp.maximum(m_y1 - m_y0, eps)
    w = jnp.maximum(m_x1 - m_x0, eps)
    ty = (yc - ya) / ha * posf
    tx = (xc - xa) / wa * posf
    th = jnp.log(h / ha) * posf
    tw = jnp.log(w / wa) * posf
    box_t = jnp.concatenate([ty, tx, th, tw], axis=0)

    boxT = jnp.transpose(box_ref[0], (1, 0))
    d = (boxT - box_t) * posf
    ad = jnp.abs(d)
    quadratic = jnp.minimum(ad, DELTA)
    linear = ad - quadratic
    huber = 0.5 * quadratic * quadratic + DELTA * linear

    logits = cls_ref[0]
    e = jnp.exp(-jnp.abs(logits))
    r = 1.0 / (1.0 + e)
    p = jnp.where(logits >= 0.0, r, 1.0 - r)
    bce = jnp.maximum(logits, 0.0) - logits * onehot + jnp.log1p(e)
    one_m_pt = p + onehot * (1.0 - 2.0 * p)
    a_t = (1.0 - ALPHA) - (1.0 - 2.0 * ALPHA) * onehot
    focal = a_t * (one_m_pt * jnp.sqrt(one_m_pt)) * bce

    acc_ref[0] += jnp.sum(focal)
    acc_ref[1] += jnp.sum(huber)
    acc_ref[2] += jnp.sum(posf)

    @pl.when(jnp.logical_and(b == nb - 1, i == ni - 1))
    def _finish():
        normalizer = acc_ref[2] + 1.0
        cls_loss = acc_ref[0] / normalizer
        box_loss = acc_ref[1] / (normalizer * 4.0)
        clsl_ref[0] = cls_loss
        boxl_ref[0] = box_loss
        total_ref[0] = cls_loss + BOX_LOSS_WEIGHT * box_loss


@jax.jit
def kernel(class_out, box_out, gt_boxes, gt_labels):
    b_dim, a_dim, c_dim = class_out.shape
    g_dim = gt_boxes.shape[1]
    nblk = a_dim // BLK
    anchors_t = jnp.asarray(_ANCHOR_BOXES_T).reshape(4, nblk, 1, BLK)
    gtl = gt_labels.reshape(b_dim, g_dim, 1)

    total, cls_loss, box_loss = pl.pallas_call(
        _loss_block,
        grid=(b_dim, nblk),
        in_specs=[
            pl.BlockSpec((1, BLK, c_dim), lambda b, i: (b, i, 0)),
            pl.BlockSpec((1, BLK, 4), lambda b, i: (b, i, 0)),
            pl.BlockSpec((4, 1, 1, BLK), lambda b, i: (0, i, 0, 0)),
            pl.BlockSpec((1, g_dim, 4), lambda b, i: (b, 0, 0)),
            pl.BlockSpec((1, g_dim, 1), lambda b, i: (b, 0, 0)),
        ],
        out_specs=[
            pl.BlockSpec(memory_space=pltpu.SMEM),
            pl.BlockSpec(memory_space=pltpu.SMEM),
            pl.BlockSpec(memory_space=pltpu.SMEM),
        ],
        out_shape=[
            jax.ShapeDtypeStruct((1,), jnp.float32),
            jax.ShapeDtypeStruct((1,), jnp.float32),
            jax.ShapeDtypeStruct((1,), jnp.float32),
        ],
        scratch_shapes=[pltpu.SMEM((3,), jnp.float32)],
    )(class_out, box_out, anchors_t, gt_boxes, gtl)

    return total[0], cls_loss[0], box_loss[0]


# speedup vs baseline: 1.4085x; 1.4085x over previous
import jax
import jax.numpy as jnp
import numpy as np
from jax import lax
from jax.experimental import pallas as pl
from jax.experimental.pallas import tpu as pltpu

MIN_LEVEL = 3
MAX_LEVEL = 7
NUM_SCALES = 3
ASPECTS = [(1.0, 1.0), (1.4, 0.7), (0.7, 1.4)]
ANCHOR_SCALE = 4.0
IMAGE_SIZE = 512
NUM_CLASSES = 90
ALPHA = 0.25
GAMMA = 1.5
DELTA = 0.1
BOX_LOSS_WEIGHT = 50.0
MATCH_THRESHOLD = 0.5
_LN2 = float(np.log(2.0))
_LOG2E = float(np.log2(np.e))


def _gen_anchor_boxes():
    boxes_all = []
    for level in range(MIN_LEVEL, MAX_LEVEL + 1):
        stride = 2 ** level
        boxes_level = []
        for octave in range(NUM_SCALES):
            for (ax, ay) in ASPECTS:
                base = ANCHOR_SCALE * stride * 2.0 ** (octave / float(NUM_SCALES))
                ah2 = base * ay / 2.0
                aw2 = base * ax / 2.0
                x = np.arange(stride / 2.0, IMAGE_SIZE, stride)
                y = np.arange(stride / 2.0, IMAGE_SIZE, stride)
                xv, yv = np.meshgrid(x, y)
                b = np.stack([yv - ah2, xv - aw2, yv + ah2, xv + aw2], axis=-1).reshape(-1, 4)
                boxes_level.append(b)
        boxes_all.append(np.stack(boxes_level, axis=1).reshape(-1, 4))
    return np.concatenate(boxes_all, axis=0).astype(np.float32)


_ANCHOR_BOXES_T = np.ascontiguousarray(_gen_anchor_boxes().T)

BLK = 4464


def _loss_block(cls_ref, boxT_ref, ancT_ref, gtb_ref, gtbT_ref, lab1h_ref, out_ref):
    b = pl.program_id(0)
    i = pl.program_id(1)
    boxT = boxT_ref[0, :, 0, 0, :]
    ancT = ancT_ref[:, 0, 0, :]

    @pl.when(jnp.logical_and(b == 0, i == 0))
    def _init():
        out_ref[0] = 0.0
        out_ref[1] = 0.0
        out_ref[2] = 0.0
        out_ref[3] = 0.0

    g = gtb_ref.shape[1]
    g_y0 = gtb_ref[0, :, 0:1]
    g_x0 = gtb_ref[0, :, 1:2]
    g_y1 = gtb_ref[0, :, 2:3]
    g_x1 = gtb_ref[0, :, 3:4]
    a_y0 = ancT[0:1, :]
    a_x0 = ancT[1:2, :]
    a_y1 = ancT[2:3, :]
    a_x1 = ancT[3:4, :]

    ymin = jnp.maximum(g_y0, a_y0)
    xmin = jnp.maximum(g_x0, a_x0)
    ymax = jnp.minimum(g_y1, a_y1)
    xmax = jnp.minimum(g_x1, a_x1)
    inter = jnp.maximum(ymax - ymin, 0.0) * jnp.maximum(xmax - xmin, 0.0)
    area_g = (g_y1 - g_y0) * (g_x1 - g_x0)
    area_a = (a_y1 - a_y0) * (a_x1 - a_x0)
    union = area_g + area_a - inter
    iou = inter / jnp.maximum(union, 1e-8)

    max_iou = jnp.max(iou, axis=0, keepdims=True)
    idx = lax.broadcasted_iota(jnp.int32, iou.shape, 0)
    arg = jnp.min(jnp.where(iou >= max_iou, idx, g), axis=0, keepdims=True)
    m = (idx == arg).astype(jnp.float32)

    posf = (max_iou >= MATCH_THRESHOLD).astype(jnp.float32)

    matched = lax.dot_general(gtbT_ref[0], m, (((1,), (0,)), ((), ())),
                              preferred_element_type=jnp.float32)
    onehot = lax.dot_general(m * posf, lab1h_ref[0], (((0,), (0,)), ((), ())),
                             preferred_element_type=jnp.float32)

    eps = 1e-8
    m_y0 = matched[0:1, :]
    m_x0 = matched[1:2, :]
    m_y1 = matched[2:3, :]
    m_x1 = matched[3:4, :]
    ya = (a_y0 + a_y1) * 0.5
    xa = (a_x0 + a_x1) * 0.5
    ha = jnp.maximum(a_y1 - a_y0, eps)
    wa = jnp.maximum(a_x1 - a_x0, eps)
    yc = (m_y0 + m_y1) * 0.5
    xc = (m_x0 + m_x1) * 0.5
    h = jnp.maximum(m_y1 - m_y0, eps)
    w = jnp.maximum(m_x1 - m_x0, eps)
    ty = (yc - ya) / ha * posf
    tx = (xc - xa) / wa * posf
    th = jnp.log(h / ha) * posf
    tw = jnp.log(w / wa) * posf
    box_t = jnp.concatenate([ty, tx, th, tw], axis=0)

    d = (boxT - box_t) * posf
    ad = jnp.abs(d)
    quadratic = jnp.minimum(ad, DELTA)
    linear = ad - quadratic
    huber = 0.5 * quadratic * quadratic + DELTA * linear

    logits = cls_ref[0]
    z2 = (logits * _LOG2E) * (2.0 * onehot - 1.0)
    e = jnp.exp2(-jnp.abs(z2))
    lg = jnp.log2(1.0 + e)
    pow_term = jnp.exp2(-1.5 * (jnp.maximum(z2, 0.0) + lg))
    neg_log_q = (jnp.maximum(-z2, 0.0) + lg) * _LN2
    a_t = (1.0 - ALPHA) - (1.0 - 2.0 * ALPHA) * onehot
    focal = a_t * pow_term * neg_log_q

    out_ref[0] += jnp.sum(focal)
    out_ref[1] += jnp.sum(huber)
    out_ref[2] += jnp.sum(posf)


@jax.jit
def kernel(class_out, box_out, gt_boxes, gt_labels):
    b_dim, a_dim, c_dim = class_out.shape
    g_dim = gt_boxes.shape[1]
    nblk = a_dim // BLK
    anchors_t = jnp.asarray(_ANCHOR_BOXES_T).reshape(4, nblk, 1, BLK)
    box_t = jnp.transpose(box_out, (0, 2, 1)).reshape(b_dim, 4, nblk, 1, BLK)
    gtb_t = jnp.transpose(gt_boxes, (0, 2, 1))
    lab1h = (gt_labels[..., None] ==
             jnp.arange(c_dim, dtype=gt_labels.dtype)).astype(jnp.float32)

    sums = pl.pallas_call(
        _loss_block,
        grid=(b_dim, nblk),
        in_specs=[
            pl.BlockSpec((1, BLK, c_dim), lambda b, i: (b, i, 0)),
            pl.BlockSpec((1, 4, 1, 1, BLK), lambda b, i: (b, 0, i, 0, 0)),
            pl.BlockSpec((4, 1, 1, BLK), lambda b, i: (0, i, 0, 0)),
            pl.BlockSpec((1, g_dim, 4), lambda b, i: (b, 0, 0)),
            pl.BlockSpec((1, 4, g_dim), lambda b, i: (b, 0, 0)),
            pl.BlockSpec((1, g_dim, c_dim), lambda b, i: (b, 0, 0)),
        ],
        out_specs=pl.BlockSpec(memory_space=pltpu.SMEM),
        out_shape=jax.ShapeDtypeStruct((4,), jnp.float32),
    )(class_out, box_t, anchors_t, gt_boxes, gtb_t, lab1h)

    normalizer = sums[2] + 1.0
    cls_loss = sums[0] / normalizer
    box_loss = sums[1] / (normalizer * 4.0)
    total = cls_loss + BOX_LOSS_WEIGHT * box_loss
    return total, cls_loss, box_loss
